# baseline (speedup 1.0000x reference)
import functools

import jax
import jax.numpy as jnp
from jax import lax
from jax.experimental import pallas as pl
from jax.experimental.pallas import tpu as pltpu
from jax.experimental.pallas import tpu_sc as plsc

E = 320000
D = 128
G = 256
NW = 32
RPW = E // NW
C = 80
NCHUNK = RPW // C
NGRP = C // 16
NV = D // 16

_mesh = plsc.VectorSubcoreMesh(core_axis_name="c", subcore_axis_name="s")


@functools.partial(
    pl.kernel,
    mesh=_mesh,
    out_type=jax.ShapeDtypeStruct((NW, G, 2 * D), jnp.float32),
    scratch_types=[
        pltpu.VMEM((2, C, D), jnp.float32),
        pltpu.VMEM((RPW,), jnp.int32),
        pltpu.VMEM((D,), jnp.float32),
        pltpu.VMEM((16,), jnp.float32),
        pltpu.VMEM((G, 2 * D), jnp.float32),
        pltpu.SemaphoreType.DMA((2,)),
    ],
)
def _sc_partials(edge_hbm, ids_hbm, w_hbm, b_hbm, out_hbm,
                 chunk_v, ids_v, w_v, b_v, acc_v, sem):
    wid = lax.axis_index("s") * 2 + lax.axis_index("c")
    base = wid * RPW

    pltpu.sync_copy(w_hbm, w_v)
    pltpu.sync_copy(b_hbm, b_v)
    pltpu.sync_copy(ids_hbm.at[pl.ds(base, RPW)], ids_v)

    zeros = jnp.zeros((16,), jnp.float32)
    ninf = jnp.full((16,), -jnp.inf, jnp.float32)

    def init_g(g, carry):
        for v in range(NV):
            acc_v[g, pl.ds(v * 16, 16)] = zeros
            acc_v[g, pl.ds(D + v * 16, 16)] = ninf
        return carry

    lax.fori_loop(0, G, init_g, 0)

    wregs = [w_v[pl.ds(v * 16, 16)] for v in range(NV)]
    bvec = b_v[...]
    lanes = lax.iota(jnp.int32, 16)
    perms_by_k = {k: jnp.bitwise_xor(lanes, k) for k in (1, 2, 4, 8)}
    masks = {k: (lanes & k) == 0 for k in (8, 4)}

    def fold(a, k):
        return a + a.at[perms_by_k[k]].get(mode="promise_in_bounds")

    def combine(a, b, k):
        return jnp.where(masks[k], fold(a, k), fold(b, k))

    def row_dot(xs):
        p = xs[0] * wregs[0]
        for v in range(1, NV):
            p = p + xs[v] * wregs[v]
        return p

    def tanh_vec(sv):
        e = jnp.exp(2.0 * sv)
        return 1.0 - 2.0 / (e + 1.0)

    def quad_weights(ps):
        t = combine(combine(ps[0], ps[1], 8), combine(ps[2], ps[3], 8), 4)
        t = fold(t, 2)
        t = fold(t, 1)
        return tanh_vec(t + bvec)

    def bcast(vec, lane):
        idx = jnp.full((16,), lane, jnp.int32)
        return vec.at[idx].get(mode="promise_in_bounds")

    QPOS = (0, 8, 4, 12)

    def row_weight(xs):
        p = row_dot(xs)
        for k in (1, 2, 4, 8):
            p = fold(p, k)
        return tanh_vec(p + bvec)

    def issue(k, buf):
        pltpu.async_copy(edge_hbm.at[pl.ds(base + k * C, C)],
                         chunk_v.at[buf], sem.at[buf])

    def drain(k, buf):
        pltpu.make_async_copy(edge_hbm.at[pl.ds(base + k * C, C)],
                              chunk_v.at[buf], sem.at[buf]).wait()

    def quad_accum(buf, i0, s, m):
        for q in range(4):
            xq = [[chunk_v[buf, i0 + 4 * q + r, pl.ds(v * 16, 16)]
                   for v in range(NV)] for r in range(4)]
            wtv = quad_weights([row_dot(xs) for xs in xq])
            for r in range(4):
                wt = bcast(wtv, QPOS[r])
                for v in range(NV):
                    s[v] = s[v] + xq[r][v] * wt
                    m[v] = jnp.maximum(m[v], xq[r][v])
        return s, m

    def flush(seg, s, m):
        for v in range(NV):
            cs = acc_v[seg, pl.ds(v * 16, 16)]
            acc_v[seg, pl.ds(v * 16, 16)] = cs + s[v]
            cm = acc_v[seg, pl.ds(D + v * 16, 16)]
            acc_v[seg, pl.ds(D + v * 16, 16)] = jnp.maximum(cm, m[v])

    def process(k, buf):
        def group_body(g, gcarry):
            idvec = ids_v[pl.ds(k * C + g * 16, 16)]
            seg0 = idvec[0]
            uniform = seg0 == idvec[15]
            i0 = g * 16

            @pl.when(uniform)
            def _fast():
                s, m = quad_accum(buf, i0, [zeros] * NV, [ninf] * NV)
                flush(seg0, s, m)

            @pl.when(jnp.logical_not(uniform))
            def _slow():
                for j in range(16):
                    xs = [chunk_v[buf, i0 + j, pl.ds(v * 16, 16)]
                          for v in range(NV)]
                    wt = row_weight(xs)
                    seg = idvec[j]
                    for v in range(NV):
                        cs = acc_v[seg, pl.ds(v * 16, 16)]
                        acc_v[seg, pl.ds(v * 16, 16)] = cs + xs[v] * wt
                        cm = acc_v[seg, pl.ds(D + v * 16, 16)]
                        acc_v[seg, pl.ds(D + v * 16, 16)] = \
                            jnp.maximum(cm, xs[v])

            return gcarry

        iv0 = ids_v[pl.ds(k * C, 16)]
        ivL = ids_v[pl.ds(k * C + C - 16, 16)]
        cseg = iv0[0]
        cuniform = cseg == ivL[15]

        @pl.when(cuniform)
        def _chunk_fast():
            def grp(g, carry):
                s, m = list(carry[:NV]), list(carry[NV:])
                s, m = quad_accum(buf, g * 16, s, m)
                return tuple(s) + tuple(m)

            fin = lax.fori_loop(0, NGRP, grp,
                                (zeros,) * NV + (ninf,) * NV)
            flush(cseg, list(fin[:NV]), list(fin[NV:]))

        @pl.when(jnp.logical_not(cuniform))
        def _per_group():
            lax.fori_loop(0, NGRP, group_body, 0)

    issue(0, 0)

    def chunk_body(k, carry):
        par = lax.rem(k, 2)

        @pl.when(k + 1 < NCHUNK)
        def _prefetch():
            issue(k + 1, 1 - par)

        drain(k, par)
        process(k, par)
        return carry

    lax.fori_loop(0, NCHUNK, chunk_body, 0)

    pltpu.sync_copy(acc_v, out_hbm.at[wid])


def _combine_body(p_ref, o_ref):
    p = p_ref[...]
    o_ref[:, :D] = jnp.sum(p[:, :, :D], axis=0)
    o_ref[:, D:] = jnp.max(p[:, :, D:], axis=0)


_combine = pl.pallas_call(
    _combine_body,
    out_shape=jax.ShapeDtypeStruct((G, 2 * D), jnp.float32),
)


def kernel(edge_feats, segment_ids, W, b):
    ids = segment_ids.astype(jnp.int32)
    wf = W.reshape(D)
    b16 = jnp.full((16,), b[0], jnp.float32)
    partial = _sc_partials(edge_feats, ids, wf, b16)
    return _combine(partial)


# speedup vs baseline: 2.8300x; 2.8300x over previous
import functools

import jax
import jax.numpy as jnp
from jax import lax
from jax.experimental import pallas as pl
from jax.experimental.pallas import tpu as pltpu
from jax.experimental.pallas import tpu_sc as plsc

E = 320000
D = 128
G = 256
NW = 32
RPW = E // NW
C = 80
NCHUNK = RPW // C
NGRP = C // 16
NV = D // 16

_mesh = plsc.VectorSubcoreMesh(core_axis_name="c", subcore_axis_name="s")


@functools.partial(
    pl.kernel,
    mesh=_mesh,
    out_type=jax.ShapeDtypeStruct((NW, G, 2 * D), jnp.float32),
    scratch_types=[
        pltpu.VMEM((2, C, D), jnp.float32),
        pltpu.VMEM((RPW,), jnp.int32),
        pltpu.VMEM((D,), jnp.float32),
        pltpu.VMEM((16,), jnp.float32),
        pltpu.VMEM((G, 2 * D), jnp.float32),
        pltpu.SemaphoreType.DMA((2,)),
    ],
)
def _sc_partials(edge_hbm, ids_hbm, w_hbm, b_hbm, out_hbm,
                 chunk_v, ids_v, w_v, b_v, acc_v, sem):
    wid = lax.axis_index("s") * 2 + lax.axis_index("c")
    base = wid * RPW

    pltpu.sync_copy(w_hbm, w_v)
    pltpu.sync_copy(b_hbm, b_v)
    pltpu.sync_copy(ids_hbm.at[pl.ds(base, RPW)], ids_v)

    zeros = jnp.zeros((16,), jnp.float32)
    ninf = jnp.full((16,), -jnp.inf, jnp.float32)

    def init_g(g, carry):
        for v in range(NV):
            acc_v[g, pl.ds(v * 16, 16)] = zeros
            acc_v[g, pl.ds(D + v * 16, 16)] = ninf
        return carry

    lax.fori_loop(0, G, init_g, 0)

    wregs = [w_v[pl.ds(v * 16, 16)] for v in range(NV)]
    bvec = b_v[...]
    lanes = lax.iota(jnp.int32, 16)
    perms_by_k = {k: jnp.bitwise_xor(lanes, k) for k in (1, 2, 4, 8)}
    masks = {k: (lanes & k) == 0 for k in (8, 4)}

    def fold(a, k):
        return a + a.at[perms_by_k[k]].get(mode="promise_in_bounds")

    def combine(a, b, k):
        return jnp.where(masks[k], fold(a, k), fold(b, k))

    def row_dot(xs):
        p = xs[0] * wregs[0]
        for v in range(1, NV):
            p = p + xs[v] * wregs[v]
        return p

    def tanh_vec(sv):
        e = jnp.exp(2.0 * sv)
        return 1.0 - 2.0 / (e + 1.0)

    def quad_weights(ps):
        t = combine(combine(ps[0], ps[1], 8), combine(ps[2], ps[3], 8), 4)
        t = fold(t, 2)
        t = fold(t, 1)
        return tanh_vec(t + bvec)

    def bcast(vec, lane):
        idx = jnp.full((16,), lane, jnp.int32)
        return vec.at[idx].get(mode="promise_in_bounds")

    QPOS = (0, 8, 4, 12)

    def row_weight(xs):
        p = row_dot(xs)
        for k in (1, 2, 4, 8):
            p = fold(p, k)
        return tanh_vec(p + bvec)

    def issue(k, buf):
        pltpu.async_copy(edge_hbm.at[pl.ds(base + k * C, C)],
                         chunk_v.at[buf], sem.at[buf])

    def drain(k, buf):
        pltpu.make_async_copy(edge_hbm.at[pl.ds(base + k * C, C)],
                              chunk_v.at[buf], sem.at[buf]).wait()

    def quad_accum(buf, i0, s, m):
        for q in range(4):
            xq = [[chunk_v[buf, i0 + 4 * q + r, pl.ds(v * 16, 16)]
                   for v in range(NV)] for r in range(4)]
            wtv = quad_weights([row_dot(xs) for xs in xq])
            for r in range(4):
                wt = bcast(wtv, QPOS[r])
                for v in range(NV):
                    s[v] = s[v] + xq[r][v] * wt
                    m[v] = jnp.maximum(m[v], xq[r][v])
        return s, m

    def flush(seg, s, m):
        for v in range(NV):
            cs = acc_v[seg, pl.ds(v * 16, 16)]
            acc_v[seg, pl.ds(v * 16, 16)] = cs + s[v]
            cm = acc_v[seg, pl.ds(D + v * 16, 16)]
            acc_v[seg, pl.ds(D + v * 16, 16)] = jnp.maximum(cm, m[v])

    def process(k, buf):
        def group_body(g, gcarry):
            idvec = ids_v[pl.ds(k * C + g * 16, 16)]
            seg0 = idvec[0]
            uniform = seg0 == idvec[15]
            i0 = g * 16

            @pl.when(uniform)
            def _fast():
                s, m = quad_accum(buf, i0, [zeros] * NV, [ninf] * NV)
                flush(seg0, s, m)

            @pl.when(jnp.logical_not(uniform))
            def _slow():
                for j in range(16):
                    xs = [chunk_v[buf, i0 + j, pl.ds(v * 16, 16)]
                          for v in range(NV)]
                    wt = row_weight(xs)
                    seg = idvec[j]
                    for v in range(NV):
                        cs = acc_v[seg, pl.ds(v * 16, 16)]
                        acc_v[seg, pl.ds(v * 16, 16)] = cs + xs[v] * wt
                        cm = acc_v[seg, pl.ds(D + v * 16, 16)]
                        acc_v[seg, pl.ds(D + v * 16, 16)] = \
                            jnp.maximum(cm, xs[v])

            return gcarry

        lax.fori_loop(0, NGRP, group_body, 0)

    issue(0, 0)

    def chunk_body(k, carry):
        par = lax.rem(k, 2)

        @pl.when(k + 1 < NCHUNK)
        def _prefetch():
            issue(k + 1, 1 - par)

        drain(k, par)
        process(k, par)
        return carry

    lax.fori_loop(0, NCHUNK, chunk_body, 0)

    pltpu.sync_copy(acc_v, out_hbm.at[wid])


def _combine_body(p_ref, o_ref):
    p = p_ref[...]
    o_ref[:, :D] = jnp.sum(p[:, :, :D], axis=0)
    o_ref[:, D:] = jnp.max(p[:, :, D:], axis=0)


_combine = pl.pallas_call(
    _combine_body,
    out_shape=jax.ShapeDtypeStruct((G, 2 * D), jnp.float32),
)


def kernel(edge_feats, segment_ids, W, b):
    ids = segment_ids.astype(jnp.int32)
    wf = W.reshape(D)
    b16 = jnp.full((16,), b[0], jnp.float32)
    partial = _sc_partials(edge_feats, ids, wf, b16)
    return _combine(partial)
